# baseline (speedup 1.0000x reference)
import functools

import jax
import jax.numpy as jnp
from jax import lax
from jax.experimental import pallas as pl
from jax.experimental.pallas import tpu as pltpu
from jax.experimental.pallas import tpu_sc as plsc

B = 256
D = 150528
K = 84
DC = D // K
NW = 32
RPW = B // NW
NBUF = 2


def _mixup_sc(x_hbm, gidx_hbm, lam_hbm, out_hbm,
              idxv, lamv, i0, i1, o0, o1,
              gsem0, gsem1, osem0, osem1):
    nc = plsc.get_sparse_core_info().num_cores
    wid = lax.axis_index("s") * nc + lax.axis_index("c")
    base = wid * RPW

    ibufs = (i0, i1)
    obufs = (o0, o1)
    gsems = (gsem0, gsem1)
    osems = (osem0, osem1)

    pltpu.sync_copy(gidx_hbm.at[wid], idxv)
    pltpu.sync_copy(lam_hbm, lamv)
    lam = lamv[...]
    one_minus_lam = 1.0 - lam

    def gather_start(c, slot):
        pltpu.make_async_copy(x_hbm.at[idxv.at[c]], ibufs[slot], gsems[slot]).start()

    def gather_wait(c, slot):
        pltpu.make_async_copy(x_hbm.at[idxv.at[c]], ibufs[slot], gsems[slot]).wait()

    def out_start(c, slot):
        pltpu.make_async_copy(obufs[slot], out_hbm.at[pl.ds(base, RPW), c],
                              osems[slot]).start()

    def out_wait(c, slot):
        pltpu.make_async_copy(obufs[slot], out_hbm.at[pl.ds(base, RPW), c],
                              osems[slot]).wait()

    for b in range(NBUF):
        gather_start(b, b)

    def step(j, carry):
        for b in range(NBUF):
            c = j * NBUF + b
            gather_wait(c, b)

            @pl.when(c >= NBUF)
            def _drain(c=c, b=b):
                out_wait(c - NBUF, b)

            ibuf, obuf = ibufs[b], obufs[b]
            for r in range(RPW):
                def col(i, acc, r=r, ibuf=ibuf, obuf=obuf):
                    sl = pl.ds(i * 16, 16)
                    obuf[r, sl] = lam * ibuf[r, sl] + one_minus_lam * ibuf[RPW + r, sl]
                    return acc
                lax.fori_loop(0, DC // 16, col, 0, unroll=4)

            out_start(c, b)

            @pl.when(c + NBUF < K)
            def _prefetch(c=c, b=b):
                gather_start(c + NBUF, b)
        return carry

    lax.fori_loop(0, K // NBUF, step, 0)

    for b in range(NBUF):
        out_wait(K - NBUF + b, b)


def kernel(inputs, index, lam):
    x = inputs.reshape(B * K, DC)
    idx32 = index.astype(jnp.int32)
    carange = jnp.arange(K, dtype=jnp.int32)
    direct = (jnp.arange(B, dtype=jnp.int32)[:, None] * K + carange[None, :])
    perm = idx32[:, None] * K + carange[None, :]
    gidx = jnp.concatenate(
        [direct.reshape(NW, RPW, K), perm.reshape(NW, RPW, K)], axis=1
    ).transpose(0, 2, 1)
    lam16 = jnp.full((16,), lam, jnp.float32)

    run = functools.partial(
        pl.kernel,
        out_type=jax.ShapeDtypeStruct((B, K, DC), jnp.float32),
        mesh=plsc.VectorSubcoreMesh(core_axis_name="c", subcore_axis_name="s"),
        scratch_types=[
            pltpu.VMEM((K, 2 * RPW), jnp.int32),
            pltpu.VMEM((16,), jnp.float32),
            pltpu.VMEM((2 * RPW, DC), jnp.float32),
            pltpu.VMEM((2 * RPW, DC), jnp.float32),
            pltpu.VMEM((RPW, DC), jnp.float32),
            pltpu.VMEM((RPW, DC), jnp.float32),
            pltpu.SemaphoreType.DMA,
            pltpu.SemaphoreType.DMA,
            pltpu.SemaphoreType.DMA,
            pltpu.SemaphoreType.DMA,
        ],
    )(_mixup_sc)
    out = run(x, gidx, lam16)
    return out.reshape(inputs.shape)


# speedup vs baseline: 1.2585x; 1.2585x over previous
import functools

import jax
import jax.numpy as jnp
from jax import lax
from jax.experimental import pallas as pl
from jax.experimental.pallas import tpu as pltpu
from jax.experimental.pallas import tpu_sc as plsc

B = 256
D = 150528
K = 84
DC = D // K
NW = 32
RPW = B // NW
NBUF = 2


def _mixup_sc(x_hbm, idx_hbm, lam_hbm, out_hbm,
              idxv, lamv, d0, d1, g0, g1, o0, o1,
              dsem0, dsem1, gsem0, gsem1, osem0, osem1):
    nc = plsc.get_sparse_core_info().num_cores
    wid = lax.axis_index("s") * nc + lax.axis_index("c")
    base = wid * RPW

    dbufs = (d0, d1)
    gbufs = (g0, g1)
    obufs = (o0, o1)
    dsems = (dsem0, dsem1)
    gsems = (gsem0, gsem1)
    osems = (osem0, osem1)

    pltpu.sync_copy(idx_hbm.at[pl.ds(base, RPW)], idxv)
    pltpu.sync_copy(lam_hbm, lamv)
    lam = lamv[...]
    one_minus_lam = 1.0 - lam

    def direct_copy(c, slot):
        return pltpu.make_async_copy(
            x_hbm.at[pl.ds(base, RPW), pl.ds(c * DC, DC)], dbufs[slot], dsems[slot])

    def gather_copy(c, slot):
        return pltpu.make_async_copy(
            x_hbm.at[idxv, pl.ds(c * DC, DC)], gbufs[slot], gsems[slot])

    def out_copy(c, slot):
        return pltpu.make_async_copy(
            obufs[slot], out_hbm.at[pl.ds(base, RPW), pl.ds(c * DC, DC)],
            osems[slot])

    for b in range(NBUF):
        direct_copy(b, b).start()
        gather_copy(b, b).start()

    def step(j, carry):
        for b in range(NBUF):
            c = j * NBUF + b
            direct_copy(c, b).wait()
            gather_copy(c, b).wait()

            @pl.when(c >= NBUF)
            def _drain(c=c, b=b):
                out_copy(c - NBUF, b).wait()

            dbuf, gbuf, obuf = dbufs[b], gbufs[b], obufs[b]
            for r in range(RPW):
                def col(i, acc, r=r, dbuf=dbuf, gbuf=gbuf, obuf=obuf):
                    sl = pl.ds(i * 16, 16)
                    obuf[r, sl] = lam * dbuf[r, sl] + one_minus_lam * gbuf[r, sl]
                    return acc
                lax.fori_loop(0, DC // 16, col, 0, unroll=8)

            out_copy(c, b).start()

            @pl.when(c + NBUF < K)
            def _prefetch(c=c, b=b):
                direct_copy(c + NBUF, b).start()
                gather_copy(c + NBUF, b).start()
        return carry

    lax.fori_loop(0, K // NBUF, step, 0)

    for b in range(NBUF):
        out_copy(K - NBUF + b, b).wait()


def kernel(inputs, index, lam):
    x = inputs.reshape(B, D)
    idx32 = index.astype(jnp.int32)
    lam16 = jnp.full((16,), lam, jnp.float32)

    run = functools.partial(
        pl.kernel,
        out_type=jax.ShapeDtypeStruct((B, D), jnp.float32),
        mesh=plsc.VectorSubcoreMesh(core_axis_name="c", subcore_axis_name="s"),
        scratch_types=[
            pltpu.VMEM((RPW,), jnp.int32),
            pltpu.VMEM((16,), jnp.float32),
            pltpu.VMEM((RPW, DC), jnp.float32),
            pltpu.VMEM((RPW, DC), jnp.float32),
            pltpu.VMEM((RPW, DC), jnp.float32),
            pltpu.VMEM((RPW, DC), jnp.float32),
            pltpu.VMEM((RPW, DC), jnp.float32),
            pltpu.VMEM((RPW, DC), jnp.float32),
            pltpu.SemaphoreType.DMA,
            pltpu.SemaphoreType.DMA,
            pltpu.SemaphoreType.DMA,
            pltpu.SemaphoreType.DMA,
            pltpu.SemaphoreType.DMA,
            pltpu.SemaphoreType.DMA,
        ],
    )(_mixup_sc)
    out = run(x, idx32, lam16)
    return out.reshape(inputs.shape)


# speedup vs baseline: 2.1754x; 1.7286x over previous
import functools

import jax
import jax.numpy as jnp
from jax import lax
from jax.experimental import pallas as pl
from jax.experimental.pallas import tpu as pltpu
from jax.experimental.pallas import tpu_sc as plsc

B = 256
D = 150528
K = 84
DC = D // K
NW = 32
RPW = B // NW
NBUF = 2


def _mixup_sc(x_hbm, idx_hbm, lam_hbm, out_hbm,
              idxv, lamv, d0, d1, g0, g1, o0, o1,
              dsem0, dsem1, gsem0, gsem1, osem0, osem1):
    nc = plsc.get_sparse_core_info().num_cores
    wid = lax.axis_index("s") * nc + lax.axis_index("c")
    base = wid * RPW

    dbufs = (d0, d1)
    gbufs = (g0, g1)
    obufs = (o0, o1)
    dsems = (dsem0, dsem1)
    gsems = (gsem0, gsem1)
    osems = (osem0, osem1)

    pltpu.sync_copy(idx_hbm.at[pl.ds(base, RPW)], idxv)
    pltpu.sync_copy(lam_hbm, lamv)
    lam = lamv[...]
    one_minus_lam = 1.0 - lam

    def direct_copy(c, slot):
        return pltpu.make_async_copy(
            x_hbm.at[pl.ds(base, RPW), pl.ds(c * DC, DC)], dbufs[slot], dsems[slot])

    def gather_copy(c, slot):
        return pltpu.make_async_copy(
            x_hbm.at[idxv, pl.ds(c * DC, DC)], gbufs[slot], gsems[slot])

    def out_copy(c, slot):
        return pltpu.make_async_copy(
            obufs[slot], out_hbm.at[pl.ds(base, RPW), pl.ds(c * DC, DC)],
            osems[slot])

    for b in range(NBUF):
        direct_copy(b, b).start()
        gather_copy(b, b).start()

    def step(j, carry):
        for b in range(NBUF):
            c = j * NBUF + b
            direct_copy(c, b).wait()
            gather_copy(c, b).wait()

            @pl.when(c >= NBUF)
            def _drain(c=c, b=b):
                out_copy(c - NBUF, b).wait()

            dbuf, gbuf, obuf = dbufs[b], gbufs[b], obufs[b]

            @plsc.parallel_loop(0, DC, step=16, unroll=4)
            def _combine(i, dbuf=dbuf, gbuf=gbuf, obuf=obuf):
                sl = pl.ds(i, 16)
                for r in range(RPW):
                    obuf[r, sl] = lam * dbuf[r, sl] + one_minus_lam * gbuf[r, sl]

            out_copy(c, b).start()

            @pl.when(c + NBUF < K)
            def _prefetch(c=c, b=b):
                direct_copy(c + NBUF, b).start()
                gather_copy(c + NBUF, b).start()
        return carry

    lax.fori_loop(0, K // NBUF, step, 0)

    for b in range(NBUF):
        out_copy(K - NBUF + b, b).wait()


def kernel(inputs, index, lam):
    x = inputs.reshape(B, D)
    idx32 = index.astype(jnp.int32)
    lam16 = jnp.full((16,), lam, jnp.float32)

    run = functools.partial(
        pl.kernel,
        out_type=jax.ShapeDtypeStruct((B, D), jnp.float32),
        mesh=plsc.VectorSubcoreMesh(core_axis_name="c", subcore_axis_name="s"),
        scratch_types=[
            pltpu.VMEM((RPW,), jnp.int32),
            pltpu.VMEM((16,), jnp.float32),
            pltpu.VMEM((RPW, DC), jnp.float32),
            pltpu.VMEM((RPW, DC), jnp.float32),
            pltpu.VMEM((RPW, DC), jnp.float32),
            pltpu.VMEM((RPW, DC), jnp.float32),
            pltpu.VMEM((RPW, DC), jnp.float32),
            pltpu.VMEM((RPW, DC), jnp.float32),
            pltpu.SemaphoreType.DMA,
            pltpu.SemaphoreType.DMA,
            pltpu.SemaphoreType.DMA,
            pltpu.SemaphoreType.DMA,
            pltpu.SemaphoreType.DMA,
            pltpu.SemaphoreType.DMA,
        ],
    )(_mixup_sc)
    out = run(x, idx32, lam16)
    return out.reshape(inputs.shape)


# speedup vs baseline: 5.1360x; 2.3610x over previous
import functools

import jax
import jax.numpy as jnp
from jax import lax
from jax.experimental import pallas as pl
from jax.experimental.pallas import tpu as pltpu
from jax.experimental.pallas import tpu_sc as plsc

B = 256
D = 150528
NW = 32
FPW = D // NW
CF = 96
K = FPW // CF
NBUF = 2
NL = B // 16


def _mixup_sc(x_hbm, idx_hbm, lam_hbm, out_hbm,
              pv, lamv, i0, i1, o0, o1,
              isem0, isem1, osem0, osem1):
    nc = plsc.get_sparse_core_info().num_cores
    wid = lax.axis_index("s") * nc + lax.axis_index("c")
    fbase = wid * FPW

    ibufs = (i0, i1)
    obufs = (o0, o1)
    isems = (isem0, isem1)
    osems = (osem0, osem1)

    pltpu.sync_copy(idx_hbm, pv)
    pltpu.sync_copy(lam_hbm, lamv)
    lam = lamv[...]
    one_minus_lam = 1.0 - lam
    pidx = [pv[pl.ds(l * 16, 16)] for l in range(NL)]

    def in_copy(c, slot):
        return pltpu.make_async_copy(
            x_hbm.at[pl.ds(fbase + c * CF, CF)], ibufs[slot], isems[slot])

    def out_copy(c, slot):
        return pltpu.make_async_copy(
            obufs[slot], out_hbm.at[pl.ds(fbase + c * CF, CF)], osems[slot])

    for b in range(NBUF):
        in_copy(b, b).start()

    def step(j, carry):
        for b in range(NBUF):
            c = j * NBUF + b
            in_copy(c, b).wait()

            @pl.when(c >= NBUF)
            def _drain(c=c, b=b):
                out_copy(c - NBUF, b).wait()

            ibuf, obuf = ibufs[b], obufs[b]

            @plsc.parallel_loop(0, CF, step=1, unroll=2)
            def _combine(f, ibuf=ibuf, obuf=obuf):
                row = jnp.broadcast_to(f, (16,))
                for l in range(NL):
                    sl = pl.ds(l * 16, 16)
                    gathered = plsc.load_gather(ibuf, [row, pidx[l]])
                    obuf[f, sl] = lam * ibuf[f, sl] + one_minus_lam * gathered

            out_copy(c, b).start()

            @pl.when(c + NBUF < K)
            def _prefetch(c=c, b=b):
                in_copy(c + NBUF, b).start()
        return carry

    lax.fori_loop(0, K // NBUF, step, 0)

    for c in range(K - K % NBUF, K):
        b = c % NBUF
        in_copy(c, b).wait()
        out_copy(c - NBUF, b).wait()
        ibuf, obuf = ibufs[b], obufs[b]

        @plsc.parallel_loop(0, CF, step=1, unroll=2)
        def _combine_tail(f, ibuf=ibuf, obuf=obuf):
            row = jnp.broadcast_to(f, (16,))
            for l in range(NL):
                sl = pl.ds(l * 16, 16)
                gathered = plsc.load_gather(ibuf, [row, pidx[l]])
                obuf[f, sl] = lam * ibuf[f, sl] + one_minus_lam * gathered

        out_copy(c, b).start()

    for b in range(NBUF):
        out_copy(b, b).wait()


def kernel(inputs, index, lam):
    xt = jnp.reshape(inputs, (B, D)).T
    idx32 = index.astype(jnp.int32)
    lam16 = jnp.full((16,), lam, jnp.float32)

    run = functools.partial(
        pl.kernel,
        out_type=jax.ShapeDtypeStruct((D, B), jnp.float32),
        mesh=plsc.VectorSubcoreMesh(core_axis_name="c", subcore_axis_name="s"),
        compiler_params=pltpu.CompilerParams(needs_layout_passes=False),
        scratch_types=[
            pltpu.VMEM((B,), jnp.int32),
            pltpu.VMEM((16,), jnp.float32),
            pltpu.VMEM((CF, B), jnp.float32),
            pltpu.VMEM((CF, B), jnp.float32),
            pltpu.VMEM((CF, B), jnp.float32),
            pltpu.VMEM((CF, B), jnp.float32),
            pltpu.SemaphoreType.DMA,
            pltpu.SemaphoreType.DMA,
            pltpu.SemaphoreType.DMA,
            pltpu.SemaphoreType.DMA,
        ],
    )(_mixup_sc)
    out_t = run(xt, idx32, lam16)
    return jnp.reshape(out_t.T, inputs.shape)


# speedup vs baseline: 5.8281x; 1.1348x over previous
import functools

import jax
import jax.numpy as jnp
from jax import lax
from jax.experimental import pallas as pl
from jax.experimental.pallas import tpu as pltpu
from jax.experimental.pallas import tpu_sc as plsc

B = 256
D = 150528
NW = 32
FPW = D // NW
CF = 112
K = FPW // CF
NBUF = 2
NL = B // 16


def _mixup_sc(x_hbm, idx_hbm, lam_hbm, out_hbm,
              pv, lamv, i0, i1, o0, o1,
              isem0, isem1, osem0, osem1):
    nc = plsc.get_sparse_core_info().num_cores
    wid = lax.axis_index("s") * nc + lax.axis_index("c")
    fbase = wid * FPW

    ibufs = (i0, i1)
    obufs = (o0, o1)
    isems = (isem0, isem1)
    osems = (osem0, osem1)

    pltpu.sync_copy(idx_hbm, pv)
    pltpu.sync_copy(lam_hbm, lamv)
    lam = lamv[...]
    one_minus_lam = 1.0 - lam
    pidx = [pv[pl.ds(l * 16, 16)] for l in range(NL)]

    def in_copy(c, slot):
        return pltpu.make_async_copy(
            x_hbm.at[pl.ds(fbase + c * CF, CF)], ibufs[slot], isems[slot])

    def out_copy(c, slot):
        return pltpu.make_async_copy(
            obufs[slot], out_hbm.at[pl.ds(fbase + c * CF, CF)], osems[slot])

    for b in range(NBUF):
        in_copy(b, b).start()

    def step(j, carry):
        for b in range(NBUF):
            c = j * NBUF + b
            in_copy(c, b).wait()

            @pl.when(c >= NBUF)
            def _drain(c=c, b=b):
                out_copy(c - NBUF, b).wait()

            ibuf, obuf = ibufs[b], obufs[b]

            @plsc.parallel_loop(0, CF, step=1, unroll=4)
            def _combine(f, ibuf=ibuf, obuf=obuf):
                row = jnp.broadcast_to(f, (16,))
                for l in range(NL):
                    sl = pl.ds(l * 16, 16)
                    gathered = plsc.load_gather(ibuf, [row, pidx[l]])
                    obuf[f, sl] = lam * ibuf[f, sl] + one_minus_lam * gathered

            out_copy(c, b).start()

            @pl.when(c + NBUF < K)
            def _prefetch(c=c, b=b):
                in_copy(c + NBUF, b).start()
        return carry

    lax.fori_loop(0, K // NBUF, step, 0)

    for c in range(K - K % NBUF, K):
        b = c % NBUF
        in_copy(c, b).wait()
        out_copy(c - NBUF, b).wait()
        ibuf, obuf = ibufs[b], obufs[b]

        @plsc.parallel_loop(0, CF, step=1, unroll=4)
        def _combine_tail(f, ibuf=ibuf, obuf=obuf):
            row = jnp.broadcast_to(f, (16,))
            for l in range(NL):
                sl = pl.ds(l * 16, 16)
                gathered = plsc.load_gather(ibuf, [row, pidx[l]])
                obuf[f, sl] = lam * ibuf[f, sl] + one_minus_lam * gathered

        out_copy(c, b).start()

    for b in range(NBUF):
        out_copy(b, b).wait()


def kernel(inputs, index, lam):
    xt = jnp.reshape(inputs, (B, D)).T
    idx32 = index.astype(jnp.int32)
    lam16 = jnp.full((16,), lam, jnp.float32)

    run = functools.partial(
        pl.kernel,
        out_type=jax.ShapeDtypeStruct((D, B), jnp.float32),
        mesh=plsc.VectorSubcoreMesh(core_axis_name="c", subcore_axis_name="s"),
        compiler_params=pltpu.CompilerParams(needs_layout_passes=False),
        scratch_types=[
            pltpu.VMEM((B,), jnp.int32),
            pltpu.VMEM((16,), jnp.float32),
            pltpu.VMEM((CF, B), jnp.float32),
            pltpu.VMEM((CF, B), jnp.float32),
            pltpu.VMEM((CF, B), jnp.float32),
            pltpu.VMEM((CF, B), jnp.float32),
            pltpu.SemaphoreType.DMA,
            pltpu.SemaphoreType.DMA,
            pltpu.SemaphoreType.DMA,
            pltpu.SemaphoreType.DMA,
        ],
    )(_mixup_sc)
    out_t = run(xt, idx32, lam16)
    return jnp.reshape(out_t.T, inputs.shape)
